# baseline (speedup 1.0000x reference)
import functools

import jax
import jax.numpy as jnp
from jax import lax
from jax.experimental import pallas as pl
from jax.experimental.pallas import tpu as pltpu
from jax.experimental.pallas import tpu_sc as plsc

_NC = 2
_NS = 16
_B = 16
_Q = 300
_C = 92
_QPAD = 320
_G = 10
_L = 16


def _sc_body(logits_hbm, boxes_hbm, params_hbm, out_hbm, lslab, bslab, oslab, pvm):
    wid = lax.axis_index("s") * _NC + lax.axis_index("c")
    b = wid // 2

    pltpu.sync_copy(params_hbm, pvm)
    pltpu.sync_copy(logits_hbm.at[wid], lslab)
    pltpu.sync_copy(boxes_hbm.at[wid], bslab)

    img_h = pvm[b, 0, :]
    img_w = pvm[b, 1, :]
    conf = pvm[b, 2, :]
    zeros = jnp.zeros((_L,), jnp.float32)

    def group(g, carry):
        def maxbody(c, m):
            return jnp.maximum(m, lslab[c, g, :])

        m = lax.fori_loop(0, _C, maxbody, jnp.full((_L,), -jnp.inf, jnp.float32))

        def smbody(c, acc):
            s, emax, lbl = acc
            e = jnp.exp(lslab[c, g, :] - m)
            upd = e > emax
            lbl = jnp.where(upd, jnp.full((_L,), c.astype(jnp.float32)), lbl)
            emax = jnp.maximum(emax, e)
            return (s + e, emax, lbl)

        s, emax, lbl = lax.fori_loop(0, _C - 1, smbody, (zeros, zeros, zeros))
        s = s + jnp.exp(lslab[_C - 1, g, :] - m)
        score = emax / s

        cx = bslab[0, g, :]
        cy = bslab[1, g, :]
        w = bslab[2, g, :]
        h = bslab[3, g, :]
        y1 = (cy - 0.5 * h) * img_h
        x1 = (cx - 0.5 * w) * img_w
        y2 = (cy + 0.5 * h) * img_h
        x2 = (cx + 0.5 * w) * img_w

        keep = score > conf
        oslab[0, g, :] = jnp.where(keep, y1, zeros)
        oslab[1, g, :] = jnp.where(keep, x1, zeros)
        oslab[2, g, :] = jnp.where(keep, y2, zeros)
        oslab[3, g, :] = jnp.where(keep, x2, zeros)
        oslab[4, g, :] = jnp.where(keep, score, zeros)
        oslab[5, g, :] = jnp.where(keep, lbl, zeros)
        return carry

    lax.fori_loop(0, _G, group, 0)
    pltpu.sync_copy(oslab, out_hbm.at[wid])


_sc_decode = functools.partial(
    pl.kernel,
    mesh=plsc.VectorSubcoreMesh(core_axis_name="c", subcore_axis_name="s"),
    out_type=jax.ShapeDtypeStruct((2 * _B, 6, _G, _L), jnp.float32),
    compiler_params=pltpu.CompilerParams(use_tc_tiling_on_sc=False),
    scratch_types=[
        pltpu.VMEM((_C, _G, _L), jnp.float32),
        pltpu.VMEM((4, _G, _L), jnp.float32),
        pltpu.VMEM((6, _G, _L), jnp.float32),
        pltpu.VMEM((_B, 3, _L), jnp.float32),
    ],
)(_sc_body)


def kernel(pred_logits, pred_boxes, target_sizes, confidence):
    pad = _QPAD - _Q
    logits_t = jnp.transpose(pred_logits, (0, 2, 1))
    logits_t = jnp.pad(logits_t, ((0, 0), (0, 0), (0, pad)))
    logits_t = logits_t.reshape(_B, _C, 2, _G, _L).transpose(0, 2, 1, 3, 4)
    logits_t = logits_t.reshape(2 * _B, _C, _G, _L)
    boxes_t = jnp.transpose(pred_boxes, (0, 2, 1))
    boxes_t = jnp.pad(boxes_t, ((0, 0), (0, 0), (0, pad)))
    boxes_t = boxes_t.reshape(_B, 4, 2, _G, _L).transpose(0, 2, 1, 3, 4)
    boxes_t = boxes_t.reshape(2 * _B, 4, _G, _L)
    conf_col = jnp.broadcast_to(
        jnp.asarray(confidence, jnp.float32).reshape(1, 1), (_B, 1)
    )
    params = jnp.concatenate([target_sizes.astype(jnp.float32), conf_col], axis=1)
    params = jnp.broadcast_to(params[:, :, None], (_B, 3, _L))
    out = _sc_decode(logits_t, boxes_t, params)
    out = out.reshape(_B, 2, 6, _G, _L).transpose(0, 2, 1, 3, 4)
    out = out.reshape(_B, 6, _QPAD).transpose(0, 2, 1)[:, :_Q, :]
    return tuple(out[i] for i in range(_B))
